# Initial kernel scaffold; baseline (speedup 1.0000x reference)
import jax
import jax.numpy as jnp
from jax.experimental import pallas as pl


def kernel(x, edge_index, W1, b1, W2, b2):
    raise NotImplementedError("write your pallas kernel here")


# speedup vs baseline: 17.3448x; 17.3448x over previous
import functools

import jax
import jax.numpy as jnp
from jax import lax
from jax.experimental import pallas as pl
from jax.experimental.pallas import tpu as pltpu
from jax.experimental.pallas import tpu_sc as plsc

N = 10000
E = 160000
D = 256
H = 32
C = 40

NC = 2
NS = 16
NW = NC * NS
CL = 128
CHUNKS = 40
EPT = CHUNKS * CL
E_PAD = NW * EPT
N_ACC = 10240
RPT = N_ACC // NS
DEGW = 16
W2P = 48
DUMMY_DST = N + 16
NBUF = 8
IDX_ROWS = NW * CHUNKS
IDX_PAD = 64
E_PAD2 = (IDX_ROWS + IDX_PAD) * CL
BN = 1000


def _zi(i):
    return i - i


def _mesh():
    return plsc.VectorSubcoreMesh(
        core_axis_name="c", subcore_axis_name="s",
        num_cores=NC, num_subcores=NS)


_SC_PARAMS = pltpu.CompilerParams(use_tc_tiling_on_sc=False)


def _sc_degree(dst2d, zeros16, ones16):
    @functools.partial(
        pl.kernel,
        out_type=jax.ShapeDtypeStruct((NC, N_ACC, DEGW), jnp.float32),
        mesh=_mesh(),
        compiler_params=_SC_PARAMS,
        scratch_types=[
            pltpu.VMEM((CHUNKS, CL), jnp.int32),
            pltpu.VMEM((CL, DEGW), jnp.float32),
            pltpu.VMEM_SHARED((N_ACC, DEGW), jnp.float32),
            pltpu.SemaphoreType.DMA,
        ],
    )
    def k(dst_hbm, zeros_hbm, ones_hbm, out_hbm, dstv, onesv, acc, ssem):
        c = lax.axis_index("c")
        s = lax.axis_index("s")
        wid = c * NS + s
        pltpu.sync_copy(zeros_hbm.at[pl.ds(s * RPT, RPT)],
                        acc.at[pl.ds(s * RPT, RPT)])
        pltpu.sync_copy(dst_hbm.at[pl.ds(wid * CHUNKS, CHUNKS)], dstv)
        pltpu.sync_copy(ones_hbm, onesv)
        plsc.subcore_barrier()

        def fire(i, j):
            pltpu.async_copy(onesv, acc.at[dstv.at[j]], ssem, add=True)
            return j + 1

        lax.fori_loop(jnp.int32(0), jnp.int32(CHUNKS), fire, jnp.int32(0))

        def drain(i, j):
            pltpu.make_async_copy(onesv, acc.at[dstv.at[j]], ssem).wait()
            return j + 1

        lax.fori_loop(jnp.int32(0), jnp.int32(CHUNKS), drain, jnp.int32(0))
        plsc.subcore_barrier()
        pltpu.sync_copy(acc.at[pl.ds(s * RPT, RPT)],
                        out_hbm.at[c, pl.ds(s * RPT, RPT)])

    return k(dst2d, zeros16, ones16)


CPT0 = 72
CPT1 = 8


def _sc_agg(src2d, dst2d, y, zeros_w, w):
    @functools.partial(
        pl.kernel,
        out_type=jax.ShapeDtypeStruct((NC, N_ACC, w), jnp.float32),
        mesh=_mesh(),
        compiler_params=_SC_PARAMS,
        scratch_types=[
            pltpu.VMEM((CPT0, CL), jnp.int32),
            pltpu.VMEM((CPT0, CL), jnp.int32),
            *([pltpu.VMEM((CL, w), jnp.float32)] * NBUF),
            pltpu.VMEM_SHARED((N_ACC, w), jnp.float32),
            *([pltpu.SemaphoreType.DMA] * (2 * NBUF)),
        ],
    )
    def k(src_hbm, dst_hbm, y_hbm, zeros_hbm, out_hbm, srcv, dstv, *rest):
        gb = rest[:NBUF]
        acc = rest[NBUF]
        gs = rest[NBUF + 1:2 * NBUF + 1]
        ss = rest[2 * NBUF + 1:]
        c = lax.axis_index("c")
        s = lax.axis_index("s")
        start = s * jnp.int32(CPT0) + c * (
            jnp.int32(NS * CPT0) - s * jnp.int32(CPT0 - CPT1))
        nloops = (jnp.int32(CPT0 // NBUF - 1)
                  - c * jnp.int32((CPT0 - CPT1) // NBUF))
        pltpu.sync_copy(zeros_hbm.at[pl.ds(s * RPT, RPT)],
                        acc.at[pl.ds(s * RPT, RPT)])
        pltpu.sync_copy(src_hbm.at[pl.ds(start, CPT0)], srcv)
        pltpu.sync_copy(dst_hbm.at[pl.ds(start, CPT0)], dstv)
        plsc.subcore_barrier()

        for b in range(NBUF):
            pltpu.async_copy(y_hbm.at[srcv.at[jnp.int32(b)]], gb[b], gs[b])

        def body(i, j):
            for b in range(NBUF):
                jb = j + b
                pltpu.make_async_copy(y_hbm.at[srcv.at[jb]],
                                      gb[b], gs[b]).wait()
                pltpu.async_copy(gb[b], acc.at[dstv.at[jb]], ss[b], add=True)
            for b in range(NBUF):
                jb = j + b
                pltpu.make_async_copy(gb[b], acc.at[dstv.at[jb]],
                                      ss[b]).wait()
                pltpu.async_copy(y_hbm.at[srcv.at[jb + NBUF]], gb[b], gs[b])
            return j + NBUF

        jf = lax.fori_loop(jnp.int32(0), nloops, body, jnp.int32(0))
        for b in range(NBUF):
            jb = jf + b
            pltpu.make_async_copy(y_hbm.at[srcv.at[jb]], gb[b], gs[b]).wait()
            pltpu.sync_copy(gb[b], acc.at[dstv.at[jb]], add=True)
        plsc.subcore_barrier()
        pltpu.sync_copy(acc.at[pl.ds(s * RPT, RPT)],
                        out_hbm.at[c, pl.ds(s * RPT, RPT)])

    return k(src2d, dst2d, y, zeros_w)


def _dinv_block(dp_ref):
    deg = dp_ref[0, :, 0:1] + dp_ref[1, :, 0:1] + 1.0
    return lax.rsqrt(deg)


def _tc_matmul1(x, w1):
    def body(x_ref, w_ref, y_ref):
        y_ref[...] = jnp.dot(x_ref[...], w_ref[...],
                             preferred_element_type=jnp.float32,
                             precision=lax.Precision.HIGHEST)

    return pl.pallas_call(
        body,
        grid=(N // BN,),
        in_specs=[
            pl.BlockSpec((BN, D), lambda i: (i, _zi(i))),
            pl.BlockSpec((D, H), lambda i: (_zi(i), _zi(i))),
        ],
        out_specs=pl.BlockSpec((BN, H), lambda i: (i, _zi(i))),
        out_shape=jax.ShapeDtypeStruct((N, H), jnp.float32),
    )(x, w1)


def _tc_stage1(xw, degp):
    def body(xw_ref, dp_ref, y_ref):
        dinv = _dinv_block(dp_ref)
        y_ref[...] = xw_ref[...] * dinv

    return pl.pallas_call(
        body,
        grid=(N // BN,),
        in_specs=[
            pl.BlockSpec((BN, H), lambda i: (i, _zi(i))),
            pl.BlockSpec((NC, BN, DEGW), lambda i: (_zi(i), i, _zi(i))),
        ],
        out_specs=pl.BlockSpec((BN, H), lambda i: (i, _zi(i))),
        out_shape=jax.ShapeDtypeStruct((N, H), jnp.float32),
    )(xw, degp)


def _tc_stage2(agg1, y1, degp, b1r, w2p):
    def body(a_ref, y_ref, dp_ref, b_ref, w_ref, o_ref):
        dinv = _dinv_block(dp_ref)
        agg = a_ref[0] + a_ref[1] + y_ref[...]
        h = jnp.maximum(agg * dinv + b_ref[...], 0.0)
        hw = jnp.dot(h, w_ref[...],
                     preferred_element_type=jnp.float32,
                     precision=lax.Precision.HIGHEST)
        o_ref[...] = hw * dinv

    return pl.pallas_call(
        body,
        grid=(N // BN,),
        in_specs=[
            pl.BlockSpec((NC, BN, H), lambda i: (_zi(i), i, _zi(i))),
            pl.BlockSpec((BN, H), lambda i: (i, _zi(i))),
            pl.BlockSpec((NC, BN, DEGW), lambda i: (_zi(i), i, _zi(i))),
            pl.BlockSpec((1, H), lambda i: (_zi(i), _zi(i))),
            pl.BlockSpec((H, W2P), lambda i: (_zi(i), _zi(i))),
        ],
        out_specs=pl.BlockSpec((BN, W2P), lambda i: (i, _zi(i))),
        out_shape=jax.ShapeDtypeStruct((N, W2P), jnp.float32),
    )(agg1, y1, degp, b1r, w2p)


def _tc_stage3(agg2, y2, degp, b2r):
    def body(a_ref, y_ref, dp_ref, b_ref, o_ref):
        dinv = _dinv_block(dp_ref)
        logits = (a_ref[0] + a_ref[1] + y_ref[...]) * dinv + b_ref[...]
        lane = lax.broadcasted_iota(jnp.int32, (BN, W2P), 1)
        valid = lane < C
        masked = jnp.where(valid, logits, -jnp.inf)
        m = jnp.max(masked, axis=1, keepdims=True)
        ex = jnp.where(valid, jnp.exp(logits - m), 0.0)
        lse = jnp.log(jnp.sum(ex, axis=1, keepdims=True))
        res = logits - m - lse
        o_ref[...] = res[:, :C]

    return pl.pallas_call(
        body,
        grid=(N // BN,),
        in_specs=[
            pl.BlockSpec((NC, BN, W2P), lambda i: (_zi(i), i, _zi(i))),
            pl.BlockSpec((BN, W2P), lambda i: (i, _zi(i))),
            pl.BlockSpec((NC, BN, DEGW), lambda i: (_zi(i), i, _zi(i))),
            pl.BlockSpec((1, W2P), lambda i: (_zi(i), _zi(i))),
        ],
        out_specs=pl.BlockSpec((BN, C), lambda i: (i, _zi(i))),
        out_shape=jax.ShapeDtypeStruct((N, C), jnp.float32),
    )(agg2, y2, degp, b2r)


def kernel(x, edge_index, W1, b1, W2, b2):
    x = x.astype(jnp.float32)
    ei = edge_index.astype(jnp.int32)
    src = jnp.concatenate(
        [ei[0], jnp.zeros((E_PAD2 - E,), jnp.int32)]
    ).reshape(IDX_ROWS + IDX_PAD, CL)
    dst = jnp.concatenate(
        [ei[1], jnp.full((E_PAD2 - E,), DUMMY_DST, jnp.int32)]
    ).reshape(IDX_ROWS + IDX_PAD, CL)

    zeros16 = jnp.zeros((N_ACC, DEGW), jnp.float32)
    zeros32 = jnp.zeros((N_ACC, H), jnp.float32)
    zeros48 = jnp.zeros((N_ACC, W2P), jnp.float32)
    ones16 = jnp.ones((CL, DEGW), jnp.float32)
    b1r = b1.astype(jnp.float32).reshape(1, H)
    b2r = jnp.pad(b2.astype(jnp.float32), (0, W2P - C)).reshape(1, W2P)
    w2p = jnp.pad(W2.astype(jnp.float32), ((0, 0), (0, W2P - C)))

    degp = _sc_degree(dst, zeros16, ones16)
    xw = _tc_matmul1(x, W1.astype(jnp.float32))
    y1 = _tc_stage1(xw, degp)
    agg1 = _sc_agg(src, dst, y1, zeros32, H)
    y2 = _tc_stage2(agg1, y1, degp, b1r, w2p)
    agg2 = _sc_agg(src, dst, y2, zeros48, W2P)
    return _tc_stage3(agg2, y2, degp, b2r)
